# baseline (speedup 1.0000x reference)
import functools

import jax
import jax.numpy as jnp
from jax import lax
from jax.experimental import pallas as pl
from jax.experimental.pallas import tpu as pltpu
from jax.experimental.pallas import tpu_sc as plsc

_B = 16384
_C = 1000
_NC = 1
_NS = 16
_NW = _NC * _NS
_RPW = _B // _NW
_HALF = _RPW // 2
_NG = _HALF // 16


def _sc_body(x2d, tgt, xt_out, neg_out,
             t_v, big_v, x0s_v, xt_v, neg_v, row_v, sem):
    wid = lax.axis_index("s") * _NC + lax.axis_index("c")
    lanes = lax.iota(jnp.int32, 16)
    zero16 = jnp.zeros((16,), jnp.int32)

    pltpu.sync_copy(tgt.at[pl.ds(wid * _RPW, _RPW)], t_v)

    for h in range(2):
        base = wid * _RPW + h * _HALF
        ho = h * _HALF

        pltpu.sync_copy(x2d.at[pl.ds(base, _HALF), pl.ds(0, 128)], big_v)

        def getx0(g, carry):
            gb = g * 16
            x0s_v[pl.ds(ho + gb, 16)] = plsc.load_gather(
                big_v, [gb + lanes, zero16])
            return carry

        lax.fori_loop(0, _NG, getx0, 0)

        def fire(g, carry):
            gb = g * 16
            t16 = t_v[pl.ds(ho + gb, 16)]
            a16 = (t16 // 128) * 128
            for l in range(16):
                a_l = pl.multiple_of(a16[l], 128)
                pltpu.async_copy(
                    x2d.at[base + gb + l, pl.ds(a_l, 128)],
                    big_v.at[gb + l], sem)
            return carry

        lax.fori_loop(0, _NG, fire, 0)

        def drain(g, carry):
            for l in range(16):
                pltpu.make_async_copy(
                    x2d.at[0, pl.ds(0, 128)], big_v.at[g * 16 + l], sem
                ).wait()
            return carry

        lax.fori_loop(0, _NG, drain, 0)

        def group(g, carry):
            gb = g * 16
            t16 = t_v[pl.ds(ho + gb, 16)]
            kidx = gb + lanes
            xt16 = plsc.load_gather(big_v, [kidx, t16 % 128])
            x016 = x0s_v[pl.ds(ho + gb, 16)]
            xt_v[pl.ds(ho + gb, 16)] = xt16
            e016 = jnp.exp(x016)
            neg_v[pl.ds(ho + gb, 16)] = e016
            nzero = plsc.all_reduce_population_count(t16 == 0)[0]

            @pl.when(nzero > 0)
            def _():
                for l in range(16):
                    t_l = t16[l]

                    @pl.when(t_l == 0)
                    def _zrow(l=l):
                        row = base + gb + l
                        pltpu.sync_copy(x2d.at[row, pl.ds(0, 896)],
                                        row_v.at[pl.ds(0, 896)])
                        s7 = pl.multiple_of(896 + t_l * 0, 128)
                        pltpu.sync_copy(x2d.at[row, pl.ds(s7, 128)],
                                        row_v.at[pl.ds(896, 128)])

                        def sexp(j, acc):
                            return acc + jnp.exp(row_v[pl.ds(j * 16, 16)])

                        accv = lax.fori_loop(0, 62, sexp,
                                             jnp.zeros((16,), jnp.float32))
                        tailv = row_v[pl.ds(992, 16)]
                        accv = accv + jnp.where(
                            lanes < 8, jnp.exp(tailv), 0.0)
                        s = jnp.float32(0.0)
                        for j in range(16):
                            s = s + accv[j]
                        cur = neg_v[pl.ds(ho + gb, 16)]
                        neg_v[pl.ds(ho + gb, 16)] = jnp.where(
                            lanes == l, s - e016[l], cur)

            return carry

        lax.fori_loop(0, _NG, group, 0)

    pltpu.sync_copy(xt_v, xt_out.at[pl.ds(wid * _RPW, _RPW)])
    pltpu.sync_copy(neg_v, neg_out.at[pl.ds(wid * _RPW, _RPW)])


def _fin_body(xt_ref, neg_ref, out_ref):
    xt = xt_ref[...]
    neg = neg_ref[...]
    out_ref[0, 0] = jnp.sum(jnp.log(neg + jnp.exp(xt)) - xt) / _B


@jax.jit
def kernel(inputs, targets):
    t32 = targets.astype(jnp.int32)

    mesh = plsc.VectorSubcoreMesh(core_axis_name="c", subcore_axis_name="s",
                                  num_cores=_NC, num_subcores=_NS)
    sc_fn = pl.kernel(
        _sc_body,
        out_type=[
            jax.ShapeDtypeStruct((_B,), jnp.float32),
            jax.ShapeDtypeStruct((_B,), jnp.float32),
        ],
        mesh=mesh,
        compiler_params=pltpu.CompilerParams(needs_layout_passes=False),
        scratch_types=[
            pltpu.VMEM((_RPW,), jnp.int32),
            pltpu.VMEM((_HALF, 128), jnp.float32),
            pltpu.VMEM((_RPW,), jnp.float32),
            pltpu.VMEM((_RPW,), jnp.float32),
            pltpu.VMEM((_RPW,), jnp.float32),
            pltpu.VMEM((1024,), jnp.float32),
            pltpu.SemaphoreType.DMA,
        ],
    )
    xt, neg = sc_fn(inputs, t32)

    loss = pl.pallas_call(
        _fin_body,
        out_specs=pl.BlockSpec(memory_space=pltpu.SMEM),
        out_shape=jax.ShapeDtypeStruct((1, 1), jnp.float32),
    )(xt.reshape(128, 128), neg.reshape(128, 128))
    return (loss[0, 0]).astype(inputs.dtype)


# speedup vs baseline: 1.4652x; 1.4652x over previous
import functools

import jax
import jax.numpy as jnp
from jax import lax
from jax.experimental import pallas as pl
from jax.experimental.pallas import tpu as pltpu
from jax.experimental.pallas import tpu_sc as plsc

_B = 16384
_C = 1000
_NC = 1
_NS = 16
_NW = _NC * _NS
_RPW = _B // _NW
_HALF = _RPW // 2
_NG = _HALF // 16


def _sc_body(xt_mat, tgt, xt_out, neg_out,
             t_v, big_v, x0r_v, xt_v, neg_v, sem):
    wid = lax.axis_index("s") * _NC + lax.axis_index("c")
    cstart = wid * _RPW
    lanes = lax.iota(jnp.int32, 16)

    pltpu.sync_copy(tgt.at[pl.ds(cstart, _RPW)], t_v)
    pltpu.sync_copy(xt_mat.at[0, pl.ds(cstart, _RPW)], x0r_v)

    for h in range(2):
        base = cstart + h * _HALF
        ho = h * _HALF

        def fire(g, carry):
            gb = g * 16
            t16 = t_v[pl.ds(ho + gb, 16)]
            iwin = pl.multiple_of(((base + gb) // 128) * 128, 128)
            for l in range(16):
                pltpu.async_copy(
                    xt_mat.at[t16[l], pl.ds(iwin, 128)],
                    big_v.at[gb + l], sem)
            return carry

        lax.fori_loop(0, _NG, fire, 0)

        def drain(g, carry):
            for l in range(16):
                pltpu.make_async_copy(
                    xt_mat.at[0, pl.ds(0, 128)], big_v.at[g * 16 + l], sem
                ).wait()
            return carry

        lax.fori_loop(0, _NG, drain, 0)

        def extract(g, carry):
            gb = g * 16
            off0 = (base + gb) - ((base + gb) // 128) * 128
            xt16 = plsc.load_gather(big_v, [gb + lanes, off0 + lanes])
            xt_v[pl.ds(ho + gb, 16)] = xt16
            neg_v[pl.ds(ho + gb, 16)] = jnp.exp(x0r_v[pl.ds(ho + gb, 16)])
            return carry

        lax.fori_loop(0, _NG, extract, 0)

        def zeros_pass(g, carry):
            gb = g * 16
            t16 = t_v[pl.ds(ho + gb, 16)]
            nzero = plsc.all_reduce_population_count(t16 == 0)[0]

            @pl.when(nzero > 0)
            def _():
                for l in range(16):
                    t_l = t16[l]

                    @pl.when(t_l == 0)
                    def _zcol(l=l):
                        i_col = base + gb + l
                        i_al = pl.multiple_of((i_col // 128) * 128, 128)
                        colv = jnp.full((16,), i_col - i_al, jnp.int32)
                        acc = jnp.zeros((16,), jnp.float32)
                        for r0 in (0, 256, 512):
                            pltpu.sync_copy(
                                xt_mat.at[pl.ds(r0, 256), pl.ds(i_al, 128)],
                                big_v.at[pl.ds(0, 256)])

                            def chunk(q, a):
                                vals = plsc.load_gather(
                                    big_v, [q * 16 + lanes, colv])
                                return a + jnp.exp(vals)

                            acc = lax.fori_loop(0, 16, chunk, acc)
                        pltpu.sync_copy(
                            xt_mat.at[pl.ds(768, 232), pl.ds(i_al, 128)],
                            big_v.at[pl.ds(0, 232)])

                        def chunk2(q, a):
                            vals = plsc.load_gather(
                                big_v, [q * 16 + lanes, colv])
                            return a + jnp.exp(vals)

                        acc = lax.fori_loop(0, 14, chunk2, acc)
                        tidx = jnp.minimum(224 + lanes, 231)
                        tvals = plsc.load_gather(big_v, [tidx, colv])
                        acc = acc + jnp.where(lanes < 8, jnp.exp(tvals), 0.0)
                        s = jnp.float32(0.0)
                        for j in range(16):
                            s = s + acc[j]
                        cur = neg_v[pl.ds(ho + gb, 16)]
                        neg_v[pl.ds(ho + gb, 16)] = jnp.where(
                            lanes == l, s - cur[l], cur)

            return carry

        lax.fori_loop(0, _NG, zeros_pass, 0)

    pltpu.sync_copy(xt_v, xt_out.at[pl.ds(cstart, _RPW)])
    pltpu.sync_copy(neg_v, neg_out.at[pl.ds(cstart, _RPW)])


def _fin_body(xt_ref, neg_ref, out_ref):
    xt = xt_ref[...]
    neg = neg_ref[...]
    out_ref[0, 0] = jnp.sum(jnp.log(neg + jnp.exp(xt)) - xt) / _B


@jax.jit
def kernel(inputs, targets):
    t32 = targets.astype(jnp.int32)
    xt_mat = inputs.T

    mesh = plsc.VectorSubcoreMesh(core_axis_name="c", subcore_axis_name="s",
                                  num_cores=_NC, num_subcores=_NS)
    sc_fn = pl.kernel(
        _sc_body,
        out_type=[
            jax.ShapeDtypeStruct((_B,), jnp.float32),
            jax.ShapeDtypeStruct((_B,), jnp.float32),
        ],
        mesh=mesh,
        compiler_params=pltpu.CompilerParams(needs_layout_passes=False),
        scratch_types=[
            pltpu.VMEM((_RPW,), jnp.int32),
            pltpu.VMEM((_HALF, 128), jnp.float32),
            pltpu.VMEM((_RPW,), jnp.float32),
            pltpu.VMEM((_RPW,), jnp.float32),
            pltpu.VMEM((_RPW,), jnp.float32),
            pltpu.SemaphoreType.DMA,
        ],
    )
    xt, neg = sc_fn(xt_mat, t32)

    loss = pl.pallas_call(
        _fin_body,
        out_specs=pl.BlockSpec(memory_space=pltpu.SMEM),
        out_shape=jax.ShapeDtypeStruct((1, 1), jnp.float32),
    )(xt.reshape(128, 128), neg.reshape(128, 128))
    return (loss[0, 0]).astype(inputs.dtype)


# speedup vs baseline: 1.6593x; 1.1325x over previous
import functools

import jax
import jax.numpy as jnp
from jax import lax
from jax.experimental import pallas as pl
from jax.experimental.pallas import tpu as pltpu
from jax.experimental.pallas import tpu_sc as plsc

_B = 16384
_C = 1000
_NC = 1
_NS = 16
_NW = _NC * _NS
_RPW = _B // _NW
_HALF = _RPW // 2
_NG = _HALF // 16


def _sc_body(xt_mat, tgt, xt_out, neg_out,
             t_v, big_v, x0r_v, xt_v, neg_v, sem):
    wid = lax.axis_index("s") * _NC + lax.axis_index("c")
    cstart = wid * _RPW
    lanes = lax.iota(jnp.int32, 16)

    pltpu.sync_copy(tgt.at[pl.ds(cstart, _RPW)], t_v)
    pltpu.sync_copy(xt_mat.at[0, pl.ds(cstart, _RPW)], x0r_v)

    for h in range(2):
        base = cstart + h * _HALF
        ho = h * _HALF

        gathers = []
        for c in range(_HALF // 128):
            iwin = pl.multiple_of(base + c * 128, 128)
            gathers.append(pltpu.async_copy(
                xt_mat.at[:, pl.ds(iwin, 128)]
                .at[t_v.at[pl.ds(ho + c * 128, 128)]],
                big_v.at[pl.ds(c * 128, 128)], sem))
        for cp in gathers:
            cp.wait()

        def extract(g, carry):
            gb = g * 16
            off0 = (base + gb) - ((base + gb) // 128) * 128
            xt16 = plsc.load_gather(big_v, [gb + lanes, off0 + lanes])
            xt_v[pl.ds(ho + gb, 16)] = xt16
            neg_v[pl.ds(ho + gb, 16)] = jnp.exp(x0r_v[pl.ds(ho + gb, 16)])
            return carry

        lax.fori_loop(0, _NG, extract, 0)

        def zeros_pass(g, carry):
            gb = g * 16
            t16 = t_v[pl.ds(ho + gb, 16)]
            nzero = plsc.all_reduce_population_count(t16 == 0)[0]

            @pl.when(nzero > 0)
            def _():
                def lane(l, carry2):
                    t_l = jnp.sum(jnp.where(lanes == l, t16, 0))

                    @pl.when(t_l == 0)
                    def _zcol():
                        i_col = base + gb + l
                        i_al = pl.multiple_of((i_col // 128) * 128, 128)
                        colv = jnp.full((16,), i_col - i_al, jnp.int32)

                        def chunk(q, a):
                            r0 = (q // 16) * 256

                            @pl.when(q % 16 == 0)
                            def _fetch():
                                pltpu.sync_copy(
                                    xt_mat.at[pl.ds(r0, 256),
                                              pl.ds(i_al, 128)],
                                    big_v.at[pl.ds(0, 256)])

                            vals = plsc.load_gather(
                                big_v, [(q % 16) * 16 + lanes, colv])
                            return a + jnp.exp(vals)

                        acc = lax.fori_loop(0, 48, chunk,
                                            jnp.zeros((16,), jnp.float32))
                        pltpu.sync_copy(
                            xt_mat.at[pl.ds(768, 232), pl.ds(i_al, 128)],
                            big_v.at[pl.ds(0, 232)])

                        def chunk2(q, a):
                            vals = plsc.load_gather(
                                big_v, [q * 16 + lanes, colv])
                            return a + jnp.exp(vals)

                        acc2 = lax.fori_loop(0, 14, chunk2, acc)
                        tidx = jnp.minimum(224 + lanes, 231)
                        tvals = plsc.load_gather(big_v, [tidx, colv])
                        acc3 = acc2 + jnp.where(lanes < 8, jnp.exp(tvals), 0.0)
                        s = jnp.sum(acc3)
                        cur = neg_v[pl.ds(ho + gb, 16)]
                        e0_l = jnp.sum(jnp.where(lanes == l, cur, 0.0))
                        neg_v[pl.ds(ho + gb, 16)] = jnp.where(
                            lanes == l, s - e0_l, cur)

                    return carry2

                lax.fori_loop(0, 16, lane, 0)

            return carry

        lax.fori_loop(0, _NG, zeros_pass, 0)

    pltpu.sync_copy(xt_v, xt_out.at[pl.ds(cstart, _RPW)])
    pltpu.sync_copy(neg_v, neg_out.at[pl.ds(cstart, _RPW)])


def _fin_body(xt_ref, neg_ref, out_ref):
    xt = xt_ref[...]
    neg = neg_ref[...]
    out_ref[0, 0] = jnp.sum(jnp.log(neg + jnp.exp(xt)) - xt) / _B


@jax.jit
def kernel(inputs, targets):
    t32 = targets.astype(jnp.int32)
    xt_mat = inputs.T

    mesh = plsc.VectorSubcoreMesh(core_axis_name="c", subcore_axis_name="s",
                                  num_cores=_NC, num_subcores=_NS)
    sc_fn = pl.kernel(
        _sc_body,
        out_type=[
            jax.ShapeDtypeStruct((_B,), jnp.float32),
            jax.ShapeDtypeStruct((_B,), jnp.float32),
        ],
        mesh=mesh,
        compiler_params=pltpu.CompilerParams(needs_layout_passes=False),
        scratch_types=[
            pltpu.VMEM((_RPW,), jnp.int32),
            pltpu.VMEM((_HALF, 128), jnp.float32),
            pltpu.VMEM((_RPW,), jnp.float32),
            pltpu.VMEM((_RPW,), jnp.float32),
            pltpu.VMEM((_RPW,), jnp.float32),
            pltpu.SemaphoreType.DMA,
        ],
    )
    xt, neg = sc_fn(xt_mat, t32)

    loss = pl.pallas_call(
        _fin_body,
        out_specs=pl.BlockSpec(memory_space=pltpu.SMEM),
        out_shape=jax.ShapeDtypeStruct((1, 1), jnp.float32),
    )(xt.reshape(128, 128), neg.reshape(128, 128))
    return (loss[0, 0]).astype(inputs.dtype)


# speedup vs baseline: 2.1150x; 1.2746x over previous
import functools

import jax
import jax.numpy as jnp
from jax import lax
from jax.experimental import pallas as pl
from jax.experimental.pallas import tpu as pltpu
from jax.experimental.pallas import tpu_sc as plsc

_B = 16384
_C = 1000
_NC = 2
_NS = 16
_NW = _NC * _NS
_RPW = _B // _NW
_HALF = _RPW // 2
_NG = _HALF // 16


def _sc_body(xt_mat, tgt, xt_out, neg_out,
             t_v, big_v, x0r_v, xt_v, neg_v, sem):
    wid = lax.axis_index("s") * _NC + lax.axis_index("c")
    cstart = wid * _RPW
    lanes = lax.iota(jnp.int32, 16)

    pltpu.sync_copy(tgt.at[pl.ds(cstart, _RPW)], t_v)
    pltpu.sync_copy(xt_mat.at[0, pl.ds(cstart, _RPW)], x0r_v)

    for h in range(2):
        base = cstart + h * _HALF
        ho = h * _HALF

        gathers = []
        for c in range(_HALF // 128):
            iwin = pl.multiple_of(base + c * 128, 128)
            gathers.append(pltpu.async_copy(
                xt_mat.at[:, pl.ds(iwin, 128)]
                .at[t_v.at[pl.ds(ho + c * 128, 128)]],
                big_v.at[pl.ds(c * 128, 128)], sem))
        for cp in gathers:
            cp.wait()

        def extract(g, carry):
            gb = g * 16
            off0 = (base + gb) - ((base + gb) // 128) * 128
            xt16 = plsc.load_gather(big_v, [gb + lanes, off0 + lanes])
            xt_v[pl.ds(ho + gb, 16)] = xt16
            neg_v[pl.ds(ho + gb, 16)] = jnp.exp(x0r_v[pl.ds(ho + gb, 16)])
            return carry

        lax.fori_loop(0, _NG, extract, 0)

        def zeros_pass(g, carry):
            gb = g * 16
            t16 = t_v[pl.ds(ho + gb, 16)]
            nzero = plsc.all_reduce_population_count(t16 == 0)[0]

            @pl.when(nzero > 0)
            def _():
                def lane(l, carry2):
                    t_l = jnp.sum(jnp.where(lanes == l, t16, 0))

                    @pl.when(t_l == 0)
                    def _zcol():
                        i_col = base + gb + l
                        i_al = pl.multiple_of((i_col // 128) * 128, 128)
                        colv = jnp.full((16,), i_col - i_al, jnp.int32)

                        def chunk(q, a):
                            r0 = (q // 16) * 256

                            @pl.when(q % 16 == 0)
                            def _fetch():
                                pltpu.sync_copy(
                                    xt_mat.at[pl.ds(r0, 256),
                                              pl.ds(i_al, 128)],
                                    big_v.at[pl.ds(0, 256)])

                            vals = plsc.load_gather(
                                big_v, [(q % 16) * 16 + lanes, colv])
                            return a + jnp.exp(vals)

                        acc = lax.fori_loop(0, 48, chunk,
                                            jnp.zeros((16,), jnp.float32))
                        pltpu.sync_copy(
                            xt_mat.at[pl.ds(768, 232), pl.ds(i_al, 128)],
                            big_v.at[pl.ds(0, 232)])

                        def chunk2(q, a):
                            vals = plsc.load_gather(
                                big_v, [q * 16 + lanes, colv])
                            return a + jnp.exp(vals)

                        acc2 = lax.fori_loop(0, 14, chunk2, acc)
                        tidx = jnp.minimum(224 + lanes, 231)
                        tvals = plsc.load_gather(big_v, [tidx, colv])
                        acc3 = acc2 + jnp.where(lanes < 8, jnp.exp(tvals), 0.0)
                        s = jnp.sum(acc3)
                        cur = neg_v[pl.ds(ho + gb, 16)]
                        e0_l = jnp.sum(jnp.where(lanes == l, cur, 0.0))
                        neg_v[pl.ds(ho + gb, 16)] = jnp.where(
                            lanes == l, s - e0_l, cur)

                    return carry2

                lax.fori_loop(0, 16, lane, 0)

            return carry

        lax.fori_loop(0, _NG, zeros_pass, 0)

    pltpu.sync_copy(xt_v, xt_out.at[pl.ds(cstart, _RPW)])
    pltpu.sync_copy(neg_v, neg_out.at[pl.ds(cstart, _RPW)])


def _fin_body(xt_ref, neg_ref, out_ref):
    xt = xt_ref[...]
    neg = neg_ref[...]
    out_ref[0, 0] = jnp.sum(jnp.log(neg + jnp.exp(xt)) - xt) / _B


@jax.jit
def kernel(inputs, targets):
    t32 = targets.astype(jnp.int32)
    xt_mat = inputs.T

    mesh = plsc.VectorSubcoreMesh(core_axis_name="c", subcore_axis_name="s",
                                  num_cores=_NC, num_subcores=_NS)
    sc_fn = pl.kernel(
        _sc_body,
        out_type=[
            jax.ShapeDtypeStruct((_B,), jnp.float32),
            jax.ShapeDtypeStruct((_B,), jnp.float32),
        ],
        mesh=mesh,
        compiler_params=pltpu.CompilerParams(needs_layout_passes=False),
        scratch_types=[
            pltpu.VMEM((_RPW,), jnp.int32),
            pltpu.VMEM((_HALF, 128), jnp.float32),
            pltpu.VMEM((_RPW,), jnp.float32),
            pltpu.VMEM((_RPW,), jnp.float32),
            pltpu.VMEM((_RPW,), jnp.float32),
            pltpu.SemaphoreType.DMA,
        ],
    )
    xt, neg = sc_fn(xt_mat, t32)

    loss = pl.pallas_call(
        _fin_body,
        out_specs=pl.BlockSpec(memory_space=pltpu.SMEM),
        out_shape=jax.ShapeDtypeStruct((1, 1), jnp.float32),
    )(xt.reshape(128, 128), neg.reshape(128, 128))
    return (loss[0, 0]).astype(inputs.dtype)
